# baseline (speedup 1.0000x reference)
import functools

import jax
import jax.numpy as jnp
from jax import lax
from jax.experimental import pallas as pl
from jax.experimental.pallas import tpu as pltpu
from jax.experimental.pallas import tpu_sc as plsc

B = 512
L = 100
NNODES = 1000
LOOKBACK = 30
NPAD = 1024
LPAD = 128
NC = 2
NS = 16
LANES = 16
NWORK = NC * NS
SPW = B // NWORK

NCHUNKS = 63
TAIL_VALID = NNODES - (NCHUNKS - 1) * LANES
LN2 = 0.6931471805599453
SQRT2 = 1.4142135623730951


def _proj_body(x_ref, w_ref, dist_ref, wv_ref, yk_ref, l0_ref):
    xb = x_ref[...]
    w = w_ref[...]
    wl = w[:, 0].reshape(1, LOOKBACK, 1)
    wr = w[:, 1].reshape(1, LOOKBACK, 1)
    y = jnp.sum(xb * wl, axis=1)
    z = jnp.sum(xb * wr, axis=1)
    w_out = wv_ref[0]
    w_dist = wv_ref[1]
    yk = w_out * y
    l0 = w_out * z + w_dist * dist_ref[...]
    pad = jnp.zeros((y.shape[0], NPAD - NNODES), jnp.float32)
    yk_ref[...] = jnp.concatenate([yk, pad], axis=1)
    l0_ref[...] = jnp.concatenate([l0, pad], axis=1).astype(jnp.bfloat16)


def _project(x, w, dist2d, wv):
    Bb = 64
    return pl.pallas_call(
        _proj_body,
        grid=(B // Bb,),
        in_specs=[
            pl.BlockSpec((Bb, LOOKBACK, NNODES), lambda b: (b, 0, 0)),
            pl.BlockSpec((LOOKBACK, 2), lambda b: (0, 0)),
            pl.BlockSpec((1, NNODES), lambda b: (0, 0)),
            pl.BlockSpec(memory_space=pltpu.SMEM),
        ],
        out_specs=[
            pl.BlockSpec((Bb, NPAD), lambda b: (b, 0)),
            pl.BlockSpec((Bb, NPAD), lambda b: (b, 0)),
        ],
        out_shape=[
            jax.ShapeDtypeStruct((B, NPAD), jnp.float32),
            jax.ShapeDtypeStruct((B, NPAD), jnp.bfloat16),
        ],
    )(x, w, dist2d, wv)


def _vlog(v):
    bits = plsc.bitcast(v, jnp.int32)
    e = jnp.right_shift(bits, 23) - 127
    mant = jnp.bitwise_or(jnp.bitwise_and(bits, 0x7FFFFF), 127 << 23)
    m = plsc.bitcast(mant, jnp.float32)
    big = m > SQRT2
    m = jnp.where(big, m * 0.5, m)
    ef = (e + jnp.where(big, 1, 0)).astype(jnp.float32)
    r = (m - 1.0) / (m + 1.0)
    r2 = r * r
    ln_m = 2.0 * r * (1.0 + r2 * (1.0 / 3.0 + r2 * (0.2 + r2 * (1.0 / 7.0 + r2 / 9.0))))
    return ln_m + ef * LN2


def _sample_fused(j, stops_v, y_v, out_v, sums_v, cnt_v):
    zero16f = jnp.zeros((LANES,), jnp.float32)
    ones16f = jnp.ones((LANES,), jnp.float32)
    one16f = jnp.full((LANES,), 1.0, jnp.float32)
    lane = lax.broadcasted_iota(jnp.int32, (LANES,), 0)
    js = jnp.zeros((LANES,), jnp.int32) + j
    edge_tail = lane < ((L - 1) - 6 * LANES)

    didx = [plsc.load_gather(stops_v, [js, lane + (c * LANES + 1)])
            for c in range(7)]
    emask = [None] * 6 + [edge_tail]
    for c in range(7):
        plsc.store_scatter(sums_v, [didx[c]], zero16f, mask=emask[c])
        plsc.store_scatter(cnt_v, [didx[c]], zero16f, mask=emask[c])
    for c in range(7):
        sidx = plsc.load_gather(stops_v, [js, lane + c * LANES])
        vals = plsc.load_gather(y_v, [js, sidx])
        plsc.addupdate_scatter(sums_v, [didx[c]], vals, mask=emask[c])
        plsc.addupdate_scatter(cnt_v, [didx[c]], ones16f, mask=emask[c])
    for c in range(7):
        sc_ = plsc.load_gather(sums_v, [didx[c]])
        cc = plsc.load_gather(cnt_v, [didx[c]])
        plsc.store_scatter(out_v, [js, didx[c]],
                           sc_ / jnp.maximum(cc, one16f), mask=emask[c])


def _seg_mean_body(stops_hbm, y_hbm, agg_hbm,
                   stops_v0, stops_v1, y_v0, y_v1, out_v0, out_v1,
                   sums_v, cnt_v, sem_in0, sem_in1, sem_out0, sem_out1):
    cid = lax.axis_index("c")
    sid = lax.axis_index("s")
    wid = sid * NC + cid
    base = wid * SPW
    GRP = 8
    zero16f = jnp.zeros((LANES,), jnp.float32)

    def start_in(b0, sv, yv, sem):
        pltpu.async_copy(stops_hbm.at[pl.ds(b0, GRP)], sv, sem)
        pltpu.async_copy(y_hbm.at[pl.ds(b0, GRP)], yv, sem)

    def wait_in(b0, sv, yv, sem):
        pltpu.make_async_copy(stops_hbm.at[pl.ds(b0, GRP)], sv, sem).wait()
        pltpu.make_async_copy(y_hbm.at[pl.ds(b0, GRP)], yv, sem).wait()

    def run_group(b0, sv, yv, ov, sem_out):
        def zero_row(j, carry):
            js = jnp.zeros((LANES,), jnp.int32) + j
            lane = lax.broadcasted_iota(jnp.int32, (LANES,), 0)
            for k in range(NPAD // LANES):
                plsc.store_scatter(ov, [js, lane + k * LANES], zero16f)
            return carry
        lax.fori_loop(0, GRP, zero_row, 0)

        def inner(j, carry):
            _sample_fused(j, sv, yv, ov, sums_v, cnt_v)
            return carry
        lax.fori_loop(0, GRP, inner, 0)
        pltpu.async_copy(ov, agg_hbm.at[pl.ds(b0, GRP)], sem_out)

    start_in(base, stops_v0, y_v0, sem_in0)
    start_in(base + GRP, stops_v1, y_v1, sem_in1)
    wait_in(base, stops_v0, y_v0, sem_in0)
    run_group(base, stops_v0, y_v0, out_v0, sem_out0)
    wait_in(base + GRP, stops_v1, y_v1, sem_in1)
    run_group(base + GRP, stops_v1, y_v1, out_v1, sem_out1)
    pltpu.make_async_copy(out_v0, agg_hbm.at[pl.ds(base, GRP)], sem_out0).wait()
    pltpu.make_async_copy(out_v1, agg_hbm.at[pl.ds(base, GRP)], sem_out1).wait()


def _seg_mean(stops_padded, yk):
    mesh = plsc.VectorSubcoreMesh(
        core_axis_name="c", subcore_axis_name="s", num_cores=NC, num_subcores=NS
    )
    f = pl.kernel(
        _seg_mean_body,
        out_type=jax.ShapeDtypeStruct((B, NPAD), jnp.float32),
        mesh=mesh,
        scratch_types=[
            pltpu.VMEM((8, LPAD), jnp.int32),
            pltpu.VMEM((8, LPAD), jnp.int32),
            pltpu.VMEM((8, NPAD), jnp.float32),
            pltpu.VMEM((8, NPAD), jnp.float32),
            pltpu.VMEM((8, NPAD), jnp.float32),
            pltpu.VMEM((8, NPAD), jnp.float32),
            pltpu.VMEM((NPAD,), jnp.float32),
            pltpu.VMEM((NPAD,), jnp.float32),
            pltpu.SemaphoreType.DMA,
            pltpu.SemaphoreType.DMA,
            pltpu.SemaphoreType.DMA,
            pltpu.SemaphoreType.DMA,
        ],
        compiler_params=pltpu.CompilerParams(needs_layout_passes=False),
    )
    return f(stops_padded, yk)


def _logits_body(agg_ref, l0_ref, mask_ref, out_ref):
    a = agg_ref[...][:, :NNODES]
    l0 = l0_ref[...][:, :NNODES].astype(jnp.float32)
    logits = a + l0
    m = jnp.max(logits, axis=1, keepdims=True)
    ex = jnp.exp(logits - m)
    lse = jnp.log(jnp.sum(ex, axis=1, keepdims=True)) + m
    logp = logits - lse
    msk = mask_ref[...] != 0
    out_ref[...] = jnp.where(msk, jnp.float32(-1e8), logp)


def _logits(agg, l0, x_mask):
    Bc = 128
    return pl.pallas_call(
        _logits_body,
        grid=(B // Bc,),
        in_specs=[
            pl.BlockSpec((Bc, NPAD), lambda b: (b, 0)),
            pl.BlockSpec((Bc, NPAD), lambda b: (b, 0)),
            pl.BlockSpec((Bc, NNODES), lambda b: (b, 0)),
        ],
        out_specs=pl.BlockSpec((Bc, NNODES), lambda b: (b, 0)),
        out_shape=jax.ShapeDtypeStruct((B, NNODES), jnp.float32),
    )(agg, l0, x_mask)


def kernel(stops, x, x_dist, x_features, x_week, x_mask, stop_emb_table,
           week_emb_table, W_l, b_l, W_r, W_fc2, b_fc2):
    w = jnp.concatenate([W_l, W_r], axis=1)
    wv = jnp.stack([W_fc2[78, 0], W_fc2[79, 0]])
    dist2d = x_dist.reshape(1, NNODES)
    yk, l0 = _project(x, w, dist2d, wv)
    stops_padded = jnp.pad(stops, ((0, 0), (0, LPAD - L)))
    agg = _seg_mean(stops_padded, yk)
    return _logits(agg, l0, x_mask)


# speedup vs baseline: 1.0043x; 1.0043x over previous
import jax
import jax.numpy as jnp
from jax import lax
from jax.experimental import pallas as pl
from jax.experimental.pallas import tpu as pltpu
from jax.experimental.pallas import tpu_sc as plsc

B = 512
L = 100
NNODES = 1000
LOOKBACK = 30
NPAD = 1024
LPAD = 128
NC = 2
NS = 16
LANES = 16
NWORK = NC * NS
SPW = B // NWORK

NCHUNKS = 63
TAIL_VALID = NNODES - (NCHUNKS - 1) * LANES
LN2 = 0.6931471805599453
SQRT2 = 1.4142135623730951


def _proj_body(x_ref, w_ref, dist_ref, wv_ref, yk_ref, l0_ref):
    xb = x_ref[...]
    w = w_ref[...]
    wl = w[:, 0].reshape(1, LOOKBACK, 1)
    wr = w[:, 1].reshape(1, LOOKBACK, 1)
    y = jnp.sum(xb * wl, axis=1)
    z = jnp.sum(xb * wr, axis=1)
    w_out = wv_ref[0]
    w_dist = wv_ref[1]
    yk = w_out * y
    l0 = w_out * z + w_dist * dist_ref[...]
    pad = jnp.zeros((y.shape[0], NPAD - NNODES), jnp.float32)
    yk_ref[...] = jnp.concatenate([yk, pad], axis=1)
    l0_ref[...] = jnp.concatenate([l0, pad], axis=1).astype(jnp.bfloat16)


def _project(x, w, dist2d, wv):
    Bb = 64
    return pl.pallas_call(
        _proj_body,
        grid=(B // Bb,),
        in_specs=[
            pl.BlockSpec((Bb, LOOKBACK, NNODES), lambda b: (b, 0, 0)),
            pl.BlockSpec((LOOKBACK, 2), lambda b: (0, 0)),
            pl.BlockSpec((1, NNODES), lambda b: (0, 0)),
            pl.BlockSpec(memory_space=pltpu.SMEM),
        ],
        out_specs=[
            pl.BlockSpec((Bb, NPAD), lambda b: (b, 0)),
            pl.BlockSpec((Bb, NPAD), lambda b: (b, 0)),
        ],
        out_shape=[
            jax.ShapeDtypeStruct((B, NPAD), jnp.float32),
            jax.ShapeDtypeStruct((B, NPAD), jnp.bfloat16),
        ],
    )(x, w, dist2d, wv)


def _vlog(v):
    bits = plsc.bitcast(v, jnp.int32)
    e = jnp.right_shift(bits, 23) - 127
    mant = jnp.bitwise_or(jnp.bitwise_and(bits, 0x7FFFFF), 127 << 23)
    m = plsc.bitcast(mant, jnp.float32)
    big = m > SQRT2
    m = jnp.where(big, m * 0.5, m)
    ef = (e + jnp.where(big, 1, 0)).astype(jnp.float32)
    r = (m - 1.0) / (m + 1.0)
    r2 = r * r
    ln_m = 2.0 * r * (1.0 + r2 * (1.0 / 3.0 + r2 * (0.2 + r2 * (1.0 / 7.0 + r2 / 9.0))))
    return ln_m + ef * LN2


def _sample_fused(j, stops_v, y_v, out_v, sums_v, cnt_v):
    zero16f = jnp.zeros((LANES,), jnp.float32)
    ones16f = jnp.ones((LANES,), jnp.float32)
    one16f = jnp.full((LANES,), 1.0, jnp.float32)
    lane = lax.broadcasted_iota(jnp.int32, (LANES,), 0)
    js = jnp.zeros((LANES,), jnp.int32) + j
    edge_tail = lane < ((L - 1) - 6 * LANES)

    didx = [plsc.load_gather(stops_v, [js, lane + (c * LANES + 1)])
            for c in range(7)]
    emask = [None] * 6 + [edge_tail]
    for c in range(7):
        plsc.store_scatter(sums_v, [didx[c]], zero16f, mask=emask[c])
        plsc.store_scatter(cnt_v, [didx[c]], zero16f, mask=emask[c])
    for c in range(7):
        sidx = plsc.load_gather(stops_v, [js, lane + c * LANES])
        vals = plsc.load_gather(y_v, [js, sidx])
        plsc.addupdate_scatter(sums_v, [didx[c]], vals, mask=emask[c])
        plsc.addupdate_scatter(cnt_v, [didx[c]], ones16f, mask=emask[c])
    for c in range(7):
        sc_ = plsc.load_gather(sums_v, [didx[c]])
        cc = plsc.load_gather(cnt_v, [didx[c]])
        plsc.store_scatter(out_v, [js, didx[c]],
                           sc_ / jnp.maximum(cc, one16f), mask=emask[c])


def _seg_mean_body(stops_hbm, y_hbm, agg_hbm,
                   stops_v0, stops_v1, y_v0, y_v1, out_v0, out_v1,
                   sums_v, cnt_v, sem_in0, sem_in1, sem_out0, sem_out1):
    cid = lax.axis_index("c")
    sid = lax.axis_index("s")
    wid = sid * NC + cid
    base = wid * SPW
    GRP = 8
    zero16f = jnp.zeros((LANES,), jnp.float32)

    def start_in(b0, sv, yv, sem):
        pltpu.async_copy(stops_hbm.at[pl.ds(b0, GRP)], sv, sem)
        pltpu.async_copy(y_hbm.at[pl.ds(b0, GRP)], yv, sem)

    def wait_in(b0, sv, yv, sem):
        pltpu.make_async_copy(stops_hbm.at[pl.ds(b0, GRP)], sv, sem).wait()
        pltpu.make_async_copy(y_hbm.at[pl.ds(b0, GRP)], yv, sem).wait()

    def run_group(b0, sv, yv, ov, sem_out):
        def zero_row(j, carry):
            js = jnp.zeros((LANES,), jnp.int32) + j
            lane = lax.broadcasted_iota(jnp.int32, (LANES,), 0)
            for k in range(NPAD // LANES):
                plsc.store_scatter(ov, [js, lane + k * LANES], zero16f)
            return carry
        lax.fori_loop(0, GRP, zero_row, 0)

        def inner(j, carry):
            _sample_fused(j, sv, yv, ov, sums_v, cnt_v)
            return carry
        lax.fori_loop(0, GRP, inner, 0)
        pltpu.async_copy(ov, agg_hbm.at[pl.ds(b0, GRP)], sem_out)

    start_in(base, stops_v0, y_v0, sem_in0)
    start_in(base + GRP, stops_v1, y_v1, sem_in1)
    wait_in(base, stops_v0, y_v0, sem_in0)
    run_group(base, stops_v0, y_v0, out_v0, sem_out0)
    wait_in(base + GRP, stops_v1, y_v1, sem_in1)
    run_group(base + GRP, stops_v1, y_v1, out_v1, sem_out1)
    pltpu.make_async_copy(out_v0, agg_hbm.at[pl.ds(base, GRP)], sem_out0).wait()
    pltpu.make_async_copy(out_v1, agg_hbm.at[pl.ds(base, GRP)], sem_out1).wait()


def _seg_mean(stops_padded, yk):
    mesh = plsc.VectorSubcoreMesh(
        core_axis_name="c", subcore_axis_name="s", num_cores=NC, num_subcores=NS
    )
    f = pl.kernel(
        _seg_mean_body,
        out_type=jax.ShapeDtypeStruct((B, NPAD), jnp.float32),
        mesh=mesh,
        scratch_types=[
            pltpu.VMEM((8, LPAD), jnp.int32),
            pltpu.VMEM((8, LPAD), jnp.int32),
            pltpu.VMEM((8, NPAD), jnp.float32),
            pltpu.VMEM((8, NPAD), jnp.float32),
            pltpu.VMEM((8, NPAD), jnp.float32),
            pltpu.VMEM((8, NPAD), jnp.float32),
            pltpu.VMEM((NPAD,), jnp.float32),
            pltpu.VMEM((NPAD,), jnp.float32),
            pltpu.SemaphoreType.DMA,
            pltpu.SemaphoreType.DMA,
            pltpu.SemaphoreType.DMA,
            pltpu.SemaphoreType.DMA,
        ],
        compiler_params=pltpu.CompilerParams(needs_layout_passes=False),
    )
    return f(stops_padded, yk)


def _logits_body(agg_ref, l0_ref, mask_ref, out_ref):
    a = agg_ref[...][:, :NNODES]
    l0 = l0_ref[...][:, :NNODES].astype(jnp.float32)
    logits = a + l0
    m = jnp.max(logits, axis=1, keepdims=True)
    ex = jnp.exp(logits - m)
    lse = jnp.log(jnp.sum(ex, axis=1, keepdims=True)) + m
    logp = logits - lse
    msk = mask_ref[...] != 0
    out_ref[...] = jnp.where(msk, jnp.float32(-1e8), logp)


def _logits(agg, l0, x_mask):
    Bc = 128
    return pl.pallas_call(
        _logits_body,
        grid=(B // Bc,),
        in_specs=[
            pl.BlockSpec((Bc, NPAD), lambda b: (b, 0)),
            pl.BlockSpec((Bc, NPAD), lambda b: (b, 0)),
            pl.BlockSpec((Bc, NNODES), lambda b: (b, 0)),
        ],
        out_specs=pl.BlockSpec((Bc, NNODES), lambda b: (b, 0)),
        out_shape=jax.ShapeDtypeStruct((B, NNODES), jnp.float32),
    )(agg, l0, x_mask)


def kernel(stops, x, x_dist, x_features, x_week, x_mask, stop_emb_table,
           week_emb_table, W_l, b_l, W_r, W_fc2, b_fc2):
    w = jnp.concatenate([W_l, W_r], axis=1)
    wv = jnp.stack([W_fc2[78, 0], W_fc2[79, 0]])
    dist2d = x_dist.reshape(1, NNODES)
    yk, l0 = _project(x, w, dist2d, wv)
    stops_padded = jnp.pad(stops, ((0, 0), (0, LPAD - L)))
    agg = _seg_mean(stops_padded, yk)
    return _logits(agg, l0, x_mask)
